# baseline (device time: 25367 ns/iter reference)
import jax
import jax.numpy as jnp
from jax import lax
from jax.experimental import pallas as pl
from jax.experimental.pallas import tpu as pltpu

N_DEV = 16
BLK = 128


def kernel(x, w_mat):
    k, m_per = x.shape
    _, n = w_mat.shape

    def body(x_ref, w_ref, out_ref, gather_ref, send_sems, recv_sems):
        me = lax.axis_index("i")

        barrier_sem = pltpu.get_barrier_semaphore()
        for s in range(1, N_DEV):
            peer = lax.rem(me + s, N_DEV)
            pl.semaphore_signal(
                barrier_sem,
                inc=1,
                device_id=(peer,),
                device_id_type=pl.DeviceIdType.MESH,
            )
        pl.semaphore_wait(barrier_sem, N_DEV - 1)

        rdmas = []
        for s in range(1, N_DEV):
            d = lax.rem(me + s, N_DEV)
            rdma = pltpu.make_async_remote_copy(
                src_ref=x_ref.at[pl.ds(d * BLK, BLK)],
                dst_ref=gather_ref.at[s],
                send_sem=send_sems.at[s],
                recv_sem=recv_sems.at[s],
                device_id=(d,),
                device_id_type=pl.DeviceIdType.MESH,
            )
            rdma.start()
            rdmas.append(rdma)

        acc = jnp.dot(
            x_ref[pl.ds(me * BLK, BLK), :],
            w_ref[pl.ds(me * BLK, BLK), :],
            preferred_element_type=jnp.float32,
        )

        for s in range(1, N_DEV):
            rdmas[s - 1].wait_recv()
            src = lax.rem(me - s + N_DEV, N_DEV)
            acc = acc + jnp.dot(
                gather_ref[s],
                w_ref[pl.ds(src * BLK, BLK), :],
                preferred_element_type=jnp.float32,
            )

        c = 0.7978845608028654
        out_ref[:, :] = 0.5 * acc * (
            1.0 + jnp.tanh(c * (acc + 0.044715 * acc * acc * acc))
        )

        for r in rdmas:
            r.wait_send()

    return pl.pallas_call(
        body,
        out_shape=jax.ShapeDtypeStruct((m_per, n), jnp.float32),
        in_specs=[
            pl.BlockSpec(memory_space=pltpu.VMEM),
            pl.BlockSpec(memory_space=pltpu.VMEM),
        ],
        out_specs=pl.BlockSpec(memory_space=pltpu.VMEM),
        scratch_shapes=[
            pltpu.VMEM((N_DEV, BLK, BLK), jnp.float32),
            pltpu.SemaphoreType.DMA((N_DEV,)),
            pltpu.SemaphoreType.DMA((N_DEV,)),
        ],
        compiler_params=pltpu.CompilerParams(collective_id=0),
    )(x, w_mat)


# device time: 24981 ns/iter; 1.0155x vs baseline; 1.0155x over previous
import jax
import jax.numpy as jnp
from jax import lax
from jax.experimental import pallas as pl
from jax.experimental.pallas import tpu as pltpu

N_DEV = 16
BLK = 128
N_GROUPS = 4
GROUP = N_DEV // N_GROUPS


def kernel(x, w_mat):
    k, m_per = x.shape
    _, n = w_mat.shape

    def body(x_ref, w_ref, out_ref, xrow_ref, send_sems, recv_sems):
        me = lax.axis_index("i")

        barrier_sem = pltpu.get_barrier_semaphore()
        for s in range(1, N_DEV):
            peer = lax.rem(me + s, N_DEV)
            pl.semaphore_signal(
                barrier_sem,
                inc=1,
                device_id=(peer,),
                device_id_type=pl.DeviceIdType.MESH,
            )
        pl.semaphore_wait(barrier_sem, N_DEV - 1)

        rdmas = []
        for s in range(1, N_DEV):
            d = lax.rem(me + s, N_DEV)
            rdma = pltpu.make_async_remote_copy(
                src_ref=x_ref.at[pl.ds(d * BLK, BLK)],
                dst_ref=xrow_ref.at[:, pl.ds(me * BLK, BLK)],
                send_sem=send_sems.at[s],
                recv_sem=recv_sems.at[me],
                device_id=(d,),
                device_id_type=pl.DeviceIdType.MESH,
            )
            rdma.start()
            rdmas.append(rdma)

        xrow_ref[:, pl.ds(me * BLK, BLK)] = x_ref[pl.ds(me * BLK, BLK), :]

        acc = jnp.zeros((m_per, n), dtype=jnp.float32)
        for g in range(N_GROUPS):
            for j in range(g * GROUP, (g + 1) * GROUP):
                recv = pltpu.make_async_remote_copy(
                    src_ref=x_ref.at[pl.ds(0, BLK)],
                    dst_ref=xrow_ref.at[:, pl.ds(j * BLK, BLK)],
                    send_sem=send_sems.at[0],
                    recv_sem=recv_sems.at[j],
                    device_id=(0,),
                    device_id_type=pl.DeviceIdType.MESH,
                )

                @pl.when(j != me)
                def _():
                    recv.wait_recv()

            acc = acc + jnp.dot(
                xrow_ref[:, pl.ds(g * GROUP * BLK, GROUP * BLK)],
                w_ref[pl.ds(g * GROUP * BLK, GROUP * BLK), :],
                preferred_element_type=jnp.float32,
            )

        c = 0.7978845608028654
        out_ref[:, :] = 0.5 * acc * (
            1.0 + jnp.tanh(c * (acc + 0.044715 * acc * acc * acc))
        )

        for r in rdmas:
            r.wait_send()

    return pl.pallas_call(
        body,
        out_shape=jax.ShapeDtypeStruct((m_per, n), jnp.float32),
        in_specs=[
            pl.BlockSpec(memory_space=pltpu.VMEM),
            pl.BlockSpec(memory_space=pltpu.VMEM),
        ],
        out_specs=pl.BlockSpec(memory_space=pltpu.VMEM),
        scratch_shapes=[
            pltpu.VMEM((m_per, k), jnp.float32),
            pltpu.SemaphoreType.DMA((N_DEV,)),
            pltpu.SemaphoreType.DMA((N_DEV,)),
        ],
        compiler_params=pltpu.CompilerParams(collective_id=0),
    )(x, w_mat)
